# baseline (speedup 1.0000x reference)
import functools

import jax
import jax.numpy as jnp
from jax import lax
from jax.experimental import pallas as pl
from jax.experimental.pallas import tpu as pltpu
from jax.experimental.pallas import tpu_sc as plsc

MOM = 0.5
C = 100
K = 8
CK = C * K
D = 256
BANK = 100000
B = 4096
MB = 512
NBLK = B // MB

NC = 2
NS = 16
NW = NC * NS
BPW = B // NW
HPW = BPW // 2


@functools.cache
def _sc_kernels():
    mesh2 = plsc.VectorSubcoreMesh(core_axis_name="c", subcore_axis_name="s",
                                   num_cores=NC, num_subcores=NS)

    @functools.partial(
        pl.kernel,
        out_type=jax.ShapeDtypeStruct((B, D), jnp.float32),
        mesh=mesh2,
        scratch_types=[
            pltpu.VMEM((BPW,), jnp.int32),
            pltpu.VMEM((BPW, D), jnp.float32),
            pltpu.SemaphoreType.DMA,
        ],
    )
    def sc_gather(bank_hbm, ids_hbm, out_hbm, idx_v, rows_v, sem):
        wid = lax.axis_index("s") * NC + lax.axis_index("c")
        base = wid * BPW
        pltpu.sync_copy(ids_hbm.at[pl.ds(base, BPW)], idx_v)
        pltpu.async_copy(bank_hbm.at[idx_v], rows_v, sem).wait()
        pltpu.sync_copy(rows_v, out_hbm.at[pl.ds(base, BPW)])

    @functools.partial(
        pl.kernel,
        out_type=jax.ShapeDtypeStruct((BANK,), jnp.int32),
        mesh=mesh2,
        scratch_types=[
            pltpu.VMEM((HPW,), jnp.int32),
            pltpu.VMEM((HPW, D), jnp.float32),
            pltpu.VMEM((BANK,), jnp.int32),
            pltpu.VMEM((B,), jnp.int32),
            pltpu.VMEM((B,), jnp.int32),
            pltpu.SemaphoreType.DMA,
        ],
        compiler_params=pltpu.CompilerParams(needs_layout_passes=False),
    )
    def sc_scatter(ids_hbm, fnew_hbm, na_hbm, assigns_hbm, bank_ref,
                   asg_out, idx_v, rows_v, asg_v, ids_v, na_v, sem):
        cid = lax.axis_index("c")
        sid = lax.axis_index("s")
        wid = sid * NC + cid

        for h in range(2):
            base = wid * BPW + h * HPW
            pltpu.sync_copy(ids_hbm.at[pl.ds(base, HPW)], idx_v)
            pltpu.sync_copy(fnew_hbm.at[pl.ds(base, HPW)], rows_v)
            pltpu.async_copy(rows_v, bank_ref.at[idx_v], sem).wait()

        @pl.when(jnp.logical_and(cid == 0, sid == 0))
        def _():
            pltpu.sync_copy(assigns_hbm, asg_v)
            pltpu.sync_copy(ids_hbm, ids_v)
            pltpu.sync_copy(na_hbm, na_v)

            def body(k, carry):
                for u in range(4):
                    off = (k * 4 + u) * 16
                    idx = ids_v[pl.ds(off, 16)]
                    v = na_v[pl.ds(off, 16)]
                    plsc.store_scatter(asg_v, [idx], v)
                return carry

            lax.fori_loop(0, B // 64, body, 0)
            pltpu.sync_copy(asg_v, asg_out)

    return sc_gather, sc_scatter


RB = 4000
GCOPY = BANK // RB


def _copy_body(bankin_ref, bankout_ref):
    bankout_ref[...] = bankin_ref[...]


def _bank_copy(bank):
    return pl.pallas_call(
        _copy_body,
        grid=(GCOPY,),
        in_specs=[pl.BlockSpec((RB, D), lambda i: (i, 0))],
        out_specs=pl.BlockSpec((RB, D), lambda i: (i, 0)),
        out_shape=jax.ShapeDtypeStruct((BANK, D), jnp.float32),
    )(bank)


def _dense_body(gath_ref, feat_ref, out_ref, cm_ref, tgt_ref,
                fnew_ref, sim_ref, na_ref, cor_ref, los_ref):
    tgt = tgt_ref[0, 0, :]
    f = feat_ref[...]
    fn1 = f / jnp.maximum(jnp.sum(jnp.abs(f), axis=1, keepdims=True), 1e-12)
    fnew = (1.0 - MOM) * gath_ref[...] + MOM * fn1
    fnew = fnew / jnp.maximum(jnp.sum(jnp.abs(fnew), axis=1, keepdims=True),
                              1e-12)
    fnew_ref[...] = fnew

    sim = lax.dot_general(cm_ref[...], fnew, (((1,), (1,)), ((), ())),
                          preferred_element_type=jnp.float32)
    row_class = lax.broadcasted_iota(jnp.int32, (CK, MB), 0) // K
    mask = (row_class != tgt[None, :]).astype(jnp.float32)
    sim = sim - 10000.0 * mask
    sim_ref[...] = sim

    best = jnp.max(sim, axis=0, keepdims=True)
    ridx = lax.broadcasted_iota(jnp.int32, (CK, MB), 0)
    na = jnp.min(jnp.where(sim == best, ridx, CK), axis=0)
    na_ref[0, 0, :] = na

    o = out_ref[...]
    obest = jnp.max(o, axis=1, keepdims=True)
    cidx = lax.broadcasted_iota(jnp.int32, (MB, C), 1)
    oam = jnp.min(jnp.where(o == obest, cidx, C), axis=1)
    cor_ref[0, 0, :] = (oam == tgt).astype(jnp.int32)

    m = jnp.max(o, axis=1, keepdims=True)
    lse = jnp.log(jnp.sum(jnp.exp(o - m), axis=1)) + m[:, 0]
    pick = jnp.sum(jnp.where(cidx == tgt[:, None], o, 0.0), axis=1)
    los_ref[0, 0, :] = lse - pick


def _tc_dense(gath, feature, out, cm2, tgt3, interpret=False):
    return pl.pallas_call(
        _dense_body,
        grid=(NBLK,),
        in_specs=[
            pl.BlockSpec((MB, D), lambda i: (i, 0)),
            pl.BlockSpec((MB, D), lambda i: (i, 0)),
            pl.BlockSpec((MB, C), lambda i: (i, 0)),
            pl.BlockSpec((CK, D), lambda i: (0, 0)),
            pl.BlockSpec((1, 1, MB), lambda i: (i, 0, 0)),
        ],
        out_specs=[
            pl.BlockSpec((MB, D), lambda i: (i, 0)),
            pl.BlockSpec((CK, MB), lambda i: (0, i)),
            pl.BlockSpec((1, 1, MB), lambda i: (i, 0, 0)),
            pl.BlockSpec((1, 1, MB), lambda i: (i, 0, 0)),
            pl.BlockSpec((1, 1, MB), lambda i: (i, 0, 0)),
        ],
        out_shape=[
            jax.ShapeDtypeStruct((B, D), jnp.float32),
            jax.ShapeDtypeStruct((CK, B), jnp.float32),
            jax.ShapeDtypeStruct((NBLK, 1, MB), jnp.int32),
            jax.ShapeDtypeStruct((NBLK, 1, MB), jnp.int32),
            jax.ShapeDtypeStruct((NBLK, 1, MB), jnp.float32),
        ],
        interpret=interpret,
    )(gath, feature, out, cm2, tgt3)


def kernel(feature, out, feature_bank, cluster_means, target, ids, assigns):
    cm2 = cluster_means.reshape(CK, D)
    tgt3 = target.astype(jnp.int32).reshape(NBLK, 1, MB)
    ids = ids.astype(jnp.int32)

    sc_gather, sc_scatter = _sc_kernels()
    gath = sc_gather(feature_bank, ids)
    bank_copy = _bank_copy(feature_bank)
    fnew, sim, na3, cor3, los3 = _tc_dense(gath, feature, out, cm2, tgt3)
    na = na3.reshape(B)
    bank_ref = jax.new_ref(bank_copy)
    asg_new = sc_scatter(ids, fnew, na, assigns, bank_ref)
    bank_new = jax.freeze(bank_ref)
    return (sim, bank_new, asg_new, na, cor3.reshape(B), los3.reshape(B))


# speedup vs baseline: 1.0139x; 1.0139x over previous
import functools

import jax
import jax.numpy as jnp
from jax import lax
from jax.experimental import pallas as pl
from jax.experimental.pallas import tpu as pltpu
from jax.experimental.pallas import tpu_sc as plsc

MOM = 0.5
C = 100
K = 8
CK = C * K
D = 256
BANK = 100000
B = 4096
MB = 512
NBLK = B // MB

NC = 2
NS = 16
NW = NC * NS
BPW = B // NW
HPW = BPW // 2


@functools.cache
def _sc_kernels():
    mesh2 = plsc.VectorSubcoreMesh(core_axis_name="c", subcore_axis_name="s",
                                   num_cores=NC, num_subcores=NS)

    @functools.partial(
        pl.kernel,
        out_type=jax.ShapeDtypeStruct((B, D), jnp.float32),
        mesh=mesh2,
        scratch_types=[
            pltpu.VMEM((BPW,), jnp.int32),
            pltpu.VMEM((BPW, D), jnp.float32),
            pltpu.SemaphoreType.DMA,
        ],
    )
    def sc_gather(bank_hbm, ids_hbm, out_hbm, idx_v, rows_v, sem):
        wid = lax.axis_index("s") * NC + lax.axis_index("c")
        base = wid * BPW
        pltpu.sync_copy(ids_hbm.at[pl.ds(base, BPW)], idx_v)
        pltpu.async_copy(bank_hbm.at[idx_v], rows_v, sem).wait()
        pltpu.sync_copy(rows_v, out_hbm.at[pl.ds(base, BPW)])

    @functools.partial(
        pl.kernel,
        out_type=jax.ShapeDtypeStruct((BANK,), jnp.int32),
        mesh=mesh2,
        scratch_types=[
            pltpu.VMEM((HPW,), jnp.int32),
            pltpu.VMEM((HPW, D), jnp.float32),
            pltpu.VMEM((BANK,), jnp.int32),
            pltpu.VMEM((B,), jnp.int32),
            pltpu.VMEM((B,), jnp.int32),
            pltpu.SemaphoreType.DMA,
        ],
        compiler_params=pltpu.CompilerParams(needs_layout_passes=False),
    )
    def sc_scatter(ids_hbm, fnew_hbm, na_hbm, assigns_hbm, bank_ref,
                   asg_out, idx_v, rows_v, asg_v, ids_v, na_v, sem):
        cid = lax.axis_index("c")
        sid = lax.axis_index("s")
        wid = sid * NC + cid

        for h in range(2):
            base = wid * BPW + h * HPW
            pltpu.sync_copy(ids_hbm.at[pl.ds(base, HPW)], idx_v)
            pltpu.sync_copy(fnew_hbm.at[pl.ds(base, HPW)], rows_v)
            pltpu.async_copy(rows_v, bank_ref.at[idx_v], sem).wait()

        @pl.when(jnp.logical_and(cid == 0, sid == 0))
        def _():
            pltpu.sync_copy(assigns_hbm, asg_v)
            pltpu.sync_copy(ids_hbm, ids_v)
            pltpu.sync_copy(na_hbm, na_v)

            def body(k, carry):
                for u in range(4):
                    off = (k * 4 + u) * 16
                    idx = ids_v[pl.ds(off, 16)]
                    v = na_v[pl.ds(off, 16)]
                    plsc.store_scatter(asg_v, [idx], v)
                return carry

            lax.fori_loop(0, B // 64, body, 0)
            pltpu.sync_copy(asg_v, asg_out)

    return sc_gather, sc_scatter


CR = 2000
NCHUNK = BANK // CR
NBUF = 10
HALF = NBUF // 2


def _copy_body(bank_any, out_any, bufs, insems, outsems):
    def cin(c, b):
        return pltpu.make_async_copy(bank_any.at[pl.ds(c * CR, CR)],
                                     bufs.at[b], insems.at[b])

    def cout(c, b):
        return pltpu.make_async_copy(bufs.at[b],
                                     out_any.at[pl.ds(c * CR, CR)],
                                     outsems.at[b])

    for b in range(NBUF):
        cin(b, b).start()
    for c in range(NCHUNK):
        cin(c, c % NBUF).wait()
        cout(c, c % NBUF).start()
        p = c - HALF
        if p >= 0 and p + NBUF < NCHUNK:
            cout(p, p % NBUF).wait()
            cin(p + NBUF, p % NBUF).start()
    for p in range(NCHUNK - NBUF, NCHUNK):
        cout(p, p % NBUF).wait()


def _bank_copy(bank):
    return pl.pallas_call(
        _copy_body,
        in_specs=[pl.BlockSpec(memory_space=pl.ANY)],
        out_specs=pl.BlockSpec(memory_space=pl.ANY),
        out_shape=jax.ShapeDtypeStruct((BANK, D), jnp.float32),
        scratch_shapes=[
            pltpu.VMEM((NBUF, CR, D), jnp.float32),
            pltpu.SemaphoreType.DMA((NBUF,)),
            pltpu.SemaphoreType.DMA((NBUF,)),
        ],
    )(bank)


def _dense_body(gath_ref, feat_ref, out_ref, cm_ref, tgt_ref,
                fnew_ref, sim_ref, na_ref, cor_ref, los_ref):
    tgt = tgt_ref[0, 0, :]
    f = feat_ref[...]
    fn1 = f / jnp.maximum(jnp.sum(jnp.abs(f), axis=1, keepdims=True), 1e-12)
    fnew = (1.0 - MOM) * gath_ref[...] + MOM * fn1
    fnew = fnew / jnp.maximum(jnp.sum(jnp.abs(fnew), axis=1, keepdims=True),
                              1e-12)
    fnew_ref[...] = fnew

    sim = lax.dot_general(cm_ref[...], fnew, (((1,), (1,)), ((), ())),
                          preferred_element_type=jnp.float32)
    row_class = lax.broadcasted_iota(jnp.int32, (CK, MB), 0) // K
    mask = (row_class != tgt[None, :]).astype(jnp.float32)
    sim = sim - 10000.0 * mask
    sim_ref[...] = sim

    best = jnp.max(sim, axis=0, keepdims=True)
    ridx = lax.broadcasted_iota(jnp.int32, (CK, MB), 0)
    na = jnp.min(jnp.where(sim == best, ridx, CK), axis=0)
    na_ref[0, 0, :] = na

    o = out_ref[...]
    obest = jnp.max(o, axis=1, keepdims=True)
    cidx = lax.broadcasted_iota(jnp.int32, (MB, C), 1)
    oam = jnp.min(jnp.where(o == obest, cidx, C), axis=1)
    cor_ref[0, 0, :] = (oam == tgt).astype(jnp.int32)

    m = jnp.max(o, axis=1, keepdims=True)
    lse = jnp.log(jnp.sum(jnp.exp(o - m), axis=1)) + m[:, 0]
    pick = jnp.sum(jnp.where(cidx == tgt[:, None], o, 0.0), axis=1)
    los_ref[0, 0, :] = lse - pick


def _tc_dense(gath, feature, out, cm2, tgt3, interpret=False):
    return pl.pallas_call(
        _dense_body,
        grid=(NBLK,),
        in_specs=[
            pl.BlockSpec((MB, D), lambda i: (i, 0)),
            pl.BlockSpec((MB, D), lambda i: (i, 0)),
            pl.BlockSpec((MB, C), lambda i: (i, 0)),
            pl.BlockSpec((CK, D), lambda i: (0, 0)),
            pl.BlockSpec((1, 1, MB), lambda i: (i, 0, 0)),
        ],
        out_specs=[
            pl.BlockSpec((MB, D), lambda i: (i, 0)),
            pl.BlockSpec((CK, MB), lambda i: (0, i)),
            pl.BlockSpec((1, 1, MB), lambda i: (i, 0, 0)),
            pl.BlockSpec((1, 1, MB), lambda i: (i, 0, 0)),
            pl.BlockSpec((1, 1, MB), lambda i: (i, 0, 0)),
        ],
        out_shape=[
            jax.ShapeDtypeStruct((B, D), jnp.float32),
            jax.ShapeDtypeStruct((CK, B), jnp.float32),
            jax.ShapeDtypeStruct((NBLK, 1, MB), jnp.int32),
            jax.ShapeDtypeStruct((NBLK, 1, MB), jnp.int32),
            jax.ShapeDtypeStruct((NBLK, 1, MB), jnp.float32),
        ],
        interpret=interpret,
    )(gath, feature, out, cm2, tgt3)


def kernel(feature, out, feature_bank, cluster_means, target, ids, assigns):
    cm2 = cluster_means.reshape(CK, D)
    tgt3 = target.astype(jnp.int32).reshape(NBLK, 1, MB)
    ids = ids.astype(jnp.int32)

    sc_gather, sc_scatter = _sc_kernels()
    gath = sc_gather(feature_bank, ids)
    bank_copy = _bank_copy(feature_bank)
    fnew, sim, na3, cor3, los3 = _tc_dense(gath, feature, out, cm2, tgt3)
    na = na3.reshape(B)
    bank_ref = jax.new_ref(bank_copy)
    asg_new = sc_scatter(ids, fnew, na, assigns, bank_ref)
    bank_new = jax.freeze(bank_ref)
    return (sim, bank_new, asg_new, na, cor3.reshape(B), los3.reshape(B))


# speedup vs baseline: 1.0704x; 1.0557x over previous
import functools

import jax
import jax.numpy as jnp
from jax import lax
from jax.experimental import pallas as pl
from jax.experimental.pallas import tpu as pltpu
from jax.experimental.pallas import tpu_sc as plsc

MOM = 0.5
C = 100
K = 8
CK = C * K
D = 256
BANK = 100000
B = 4096
MB = 512
NBLK = B // MB

NC = 2
NS = 16
NW = NC * NS
BPW = B // NW
HPW = BPW // 2


@functools.cache
def _sc_kernels():
    mesh2 = plsc.VectorSubcoreMesh(core_axis_name="c", subcore_axis_name="s",
                                   num_cores=NC, num_subcores=NS)

    @functools.partial(
        pl.kernel,
        out_type=jax.ShapeDtypeStruct((B, D), jnp.float32),
        mesh=mesh2,
        scratch_types=[
            pltpu.VMEM((BPW,), jnp.int32),
            pltpu.VMEM((BPW, D), jnp.float32),
            pltpu.SemaphoreType.DMA,
        ],
    )
    def sc_gather(bank_hbm, ids_hbm, out_hbm, idx_v, rows_v, sem):
        wid = lax.axis_index("s") * NC + lax.axis_index("c")
        base = wid * BPW
        pltpu.sync_copy(ids_hbm.at[pl.ds(base, BPW)], idx_v)
        pltpu.async_copy(bank_hbm.at[idx_v], rows_v, sem).wait()
        pltpu.sync_copy(rows_v, out_hbm.at[pl.ds(base, BPW)])

    @functools.partial(
        pl.kernel,
        out_type=jax.ShapeDtypeStruct((BANK,), jnp.int32),
        mesh=mesh2,
        scratch_types=[
            pltpu.VMEM((HPW,), jnp.int32),
            pltpu.VMEM((8,), jnp.int32),
            pltpu.VMEM((16,), jnp.int32),
            pltpu.VMEM((HPW, D), jnp.float32),
            pltpu.VMEM((BANK,), jnp.int32),
            pltpu.VMEM((B,), jnp.int32),
            pltpu.VMEM((B,), jnp.int32),
            pltpu.SemaphoreType.DMA,
        ],
        compiler_params=pltpu.CompilerParams(needs_layout_passes=False),
    )
    def sc_scatter(ids_hbm, fnew_hbm, na_hbm, assigns_hbm, bank_ref,
                   asg_out, idx64, idx8, idx16, rows_v, asg_v, ids_v, na_v,
                   sem):
        cid = lax.axis_index("c")
        sid = lax.axis_index("s")
        wid = sid * NC + cid

        @pl.when(jnp.logical_and(wid >= 1, wid <= 30))
        def _():
            base = (wid - 1) * 136
            for off, sz, idxr in ((0, HPW, idx64), (HPW, HPW, idx64),
                                  (2 * HPW, 8, idx8)):
                pltpu.sync_copy(ids_hbm.at[pl.ds(base + off, sz)], idxr)
                pltpu.sync_copy(fnew_hbm.at[pl.ds(base + off, sz)],
                                rows_v.at[pl.ds(0, sz)])
                pltpu.async_copy(rows_v.at[pl.ds(0, sz)],
                                 bank_ref.at[idxr], sem).wait()

        @pl.when(wid == 31)
        def _():
            pltpu.sync_copy(ids_hbm.at[pl.ds(30 * 136, 16)], idx16)
            pltpu.sync_copy(fnew_hbm.at[pl.ds(30 * 136, 16)],
                            rows_v.at[pl.ds(0, 16)])
            pltpu.async_copy(rows_v.at[pl.ds(0, 16)],
                             bank_ref.at[idx16], sem).wait()

        @pl.when(wid == 0)
        def _():
            pltpu.sync_copy(assigns_hbm, asg_v)
            pltpu.sync_copy(ids_hbm, ids_v)
            pltpu.sync_copy(na_hbm, na_v)

            def body(k, carry):
                for u in range(4):
                    off = (k * 4 + u) * 16
                    idx = ids_v[pl.ds(off, 16)]
                    v = na_v[pl.ds(off, 16)]
                    plsc.store_scatter(asg_v, [idx], v)
                return carry

            lax.fori_loop(0, B // 64, body, 0)
            pltpu.sync_copy(asg_v, asg_out)

    return sc_gather, sc_scatter


RB = 4000
GCOPY = BANK // RB


def _dense_body(bankin_ref, gath_ref, feat_ref, out_ref, cm_ref, tgt_ref,
                bankout_ref, fnew_ref, sim_ref, na_ref, cor_ref, los_ref):
    i = pl.program_id(0)
    bankout_ref[...] = bankin_ref[...]

    @pl.when(i < NBLK)
    def _dense():
        _dense_compute(gath_ref, feat_ref, out_ref, cm_ref, tgt_ref,
                       fnew_ref, sim_ref, na_ref, cor_ref, los_ref)


def _dense_compute(gath_ref, feat_ref, out_ref, cm_ref, tgt_ref,
                   fnew_ref, sim_ref, na_ref, cor_ref, los_ref):
    tgt = tgt_ref[0, 0, :]
    f = feat_ref[...]
    fn1 = f / jnp.maximum(jnp.sum(jnp.abs(f), axis=1, keepdims=True), 1e-12)
    fnew = (1.0 - MOM) * gath_ref[...] + MOM * fn1
    fnew = fnew / jnp.maximum(jnp.sum(jnp.abs(fnew), axis=1, keepdims=True),
                              1e-12)
    fnew_ref[...] = fnew

    sim = lax.dot_general(cm_ref[...], fnew, (((1,), (1,)), ((), ())),
                          preferred_element_type=jnp.float32)
    row_class = lax.broadcasted_iota(jnp.int32, (CK, MB), 0) // K
    mask = (row_class != tgt[None, :]).astype(jnp.float32)
    sim = sim - 10000.0 * mask
    sim_ref[...] = sim

    best = jnp.max(sim, axis=0, keepdims=True)
    ridx = lax.broadcasted_iota(jnp.int32, (CK, MB), 0)
    na = jnp.min(jnp.where(sim == best, ridx, CK), axis=0)
    na_ref[0, 0, :] = na

    o = out_ref[...]
    obest = jnp.max(o, axis=1, keepdims=True)
    cidx = lax.broadcasted_iota(jnp.int32, (MB, C), 1)
    oam = jnp.min(jnp.where(o == obest, cidx, C), axis=1)
    cor_ref[0, 0, :] = (oam == tgt).astype(jnp.int32)

    m = jnp.max(o, axis=1, keepdims=True)
    lse = jnp.log(jnp.sum(jnp.exp(o - m), axis=1)) + m[:, 0]
    pick = jnp.sum(jnp.where(cidx == tgt[:, None], o, 0.0), axis=1)
    los_ref[0, 0, :] = lse - pick


def _tc_dense(bank, gath, feature, out, cm2, tgt3, interpret=False):
    def dmap(i):
        return (jnp.minimum(i, NBLK - 1), 0)

    def dmap3(i):
        return (jnp.minimum(i, NBLK - 1), 0, 0)

    return pl.pallas_call(
        _dense_body,
        grid=(GCOPY,),
        in_specs=[
            pl.BlockSpec((RB, D), lambda i: (i, 0)),
            pl.BlockSpec((MB, D), dmap),
            pl.BlockSpec((MB, D), dmap),
            pl.BlockSpec((MB, C), dmap),
            pl.BlockSpec((CK, D), lambda i: (0, 0)),
            pl.BlockSpec((1, 1, MB), dmap3),
        ],
        out_specs=[
            pl.BlockSpec((RB, D), lambda i: (i, 0)),
            pl.BlockSpec((MB, D), dmap),
            pl.BlockSpec((CK, MB), lambda i: (0, jnp.minimum(i, NBLK - 1))),
            pl.BlockSpec((1, 1, MB), dmap3),
            pl.BlockSpec((1, 1, MB), dmap3),
            pl.BlockSpec((1, 1, MB), dmap3),
        ],
        out_shape=[
            jax.ShapeDtypeStruct((BANK, D), jnp.float32),
            jax.ShapeDtypeStruct((B, D), jnp.float32),
            jax.ShapeDtypeStruct((CK, B), jnp.float32),
            jax.ShapeDtypeStruct((NBLK, 1, MB), jnp.int32),
            jax.ShapeDtypeStruct((NBLK, 1, MB), jnp.int32),
            jax.ShapeDtypeStruct((NBLK, 1, MB), jnp.float32),
        ],
        interpret=interpret,
    )(bank, gath, feature, out, cm2, tgt3)


def kernel(feature, out, feature_bank, cluster_means, target, ids, assigns):
    cm2 = cluster_means.reshape(CK, D)
    tgt3 = target.astype(jnp.int32).reshape(NBLK, 1, MB)
    ids = ids.astype(jnp.int32)

    sc_gather, sc_scatter = _sc_kernels()
    gath = sc_gather(feature_bank, ids)
    bank_copy, fnew, sim, na3, cor3, los3 = _tc_dense(
        feature_bank, gath, feature, out, cm2, tgt3)
    na = na3.reshape(B)
    bank_ref = jax.new_ref(bank_copy)
    asg_new = sc_scatter(ids, fnew, na, assigns, bank_ref)
    bank_new = jax.freeze(bank_ref)
    return (sim, bank_new, asg_new, na, cor3.reshape(B), los3.reshape(B))
